# baseline (speedup 1.0000x reference)
import jax
import jax.numpy as jnp
from jax import lax
from jax.experimental import pallas as pl
from jax.experimental.pallas import tpu as pltpu
from jax.experimental.pallas import tpu_sc as plsc

_S = 24
_N = 577
_D = 64
_NV = _D // 16
_NC = 2
_NS = 16
_NW = _NC * _NS
_RPW = 18
_HA = 288


def _rp2d_body(v_hbm, h_hbm, out_hbm, v_vm, h_vm, row_vm, cls_vm,
               sem_a, sem_b, sem_c):
    w = lax.axis_index("s") * _NC + lax.axis_index("c")
    pltpu.sync_copy(v_hbm, v_vm)
    pltpu.sync_copy(h_hbm, h_vm)

    cls_vec = [v_vm[0, pl.ds(d * 16, 16)] + h_vm[0, pl.ds(d * 16, 16)]
               for d in range(_NV)]

    for r in range(_S):
        for d in range(_NV):
            cls_vm[r, pl.ds(d * 16, 16)] = cls_vec[d]

    @pl.when(w < _S)
    def _():
        pltpu.async_copy(cls_vm, out_hbm.at[0, pl.ds(w * _S, _S)], sem_c)

    @pl.when(w == _S)
    def _():
        pltpu.async_copy(cls_vm.at[pl.ds(0, 1)],
                         out_hbm.at[0, pl.ds(_N - 1, 1)], sem_c)

    def wait_half(sem, lo, n):
        pltpu.make_async_copy(
            row_vm.at[pl.ds(lo, n)], out_hbm.at[0, pl.ds(lo, n)], sem).wait()

    def _slice_starts(q):
        qb = (q - 1) // _S
        qr = (q - 1) % _S
        return (_S + 1) - qb, (_S + 1) - qr

    def _emit_group(vb, hb, kb0, nkb, kr_cut=None):
        vv = [[v_vm[vb + (kb0 + i), pl.ds(d * 16, 16)]
               for d in range(_NV)] for i in range(nkb)]
        for kr in range(_S):
            hrow = hb + kr
            for d in range(_NV):
                h = h_vm[hrow, pl.ds(d * 16, 16)]
                for i in range(nkb):
                    if kr_cut is not None and i == nkb - 1 and kr >= kr_cut:
                        continue
                    r = 1 + (kb0 + i) * _S + kr
                    row_vm[r, pl.ds(d * 16, 16)] = vv[i][d] + h

    def _emit_group_dyn(vb, hb, kb0):
        for i in range(4):
            vvi = [v_vm[vb + kb0 + i, pl.ds(d * 16, 16)]
                   for d in range(_NV)]
            rb = 1 + (kb0 + i) * _S
            for kr in range(_S):
                hrow = hb + kr
                for d in range(_NV):
                    row_vm[rb + kr, pl.ds(d * 16, 16)] = (
                        vvi[d] + h_vm[hrow, pl.ds(d * 16, 16)])

    def do_row(j, carry):
        plsc.subcore_barrier()
        q = 1 + w + _NW * j
        vb, hb = _slice_starts(q)

        @pl.when(j >= 1)
        def _():
            wait_half(sem_a, 0, _HA)

        for d in range(_NV):
            row_vm[0, pl.ds(d * 16, 16)] = cls_vec[d]

        @plsc.parallel_loop(0, 2, 1, unroll=1)
        def _(g):
            _emit_group_dyn(vb, hb, g * 4)

        _emit_group(vb, hb, 8, 4, kr_cut=_S - 1)

        pltpu.async_copy(row_vm.at[pl.ds(0, _HA)],
                         out_hbm.at[q, pl.ds(0, _HA)], sem_a)

        @pl.when(j >= 1)
        def _():
            wait_half(sem_b, _HA, _N - _HA)

        for d in range(_NV):
            row_vm[_HA, pl.ds(d * 16, 16)] = (
                v_vm[vb + 11, pl.ds(d * 16, 16)]
                + h_vm[hb + _S - 1, pl.ds(d * 16, 16)])
        @plsc.parallel_loop(3, 6, 1, unroll=1)
        def _(g):
            _emit_group_dyn(vb, hb, g * 4)

        pltpu.async_copy(row_vm.at[pl.ds(_HA, _N - _HA)],
                         out_hbm.at[q, pl.ds(_HA, _N - _HA)], sem_b)

        return carry

    lax.fori_loop(0, _RPW, do_row, 0)
    wait_half(sem_a, 0, _HA)
    wait_half(sem_b, _HA, _N - _HA)

    @pl.when(w < _S)
    def _():
        pltpu.make_async_copy(
            cls_vm, out_hbm.at[0, pl.ds(0, _S)], sem_c).wait()

    @pl.when(w == _S)
    def _():
        pltpu.make_async_copy(
            cls_vm.at[pl.ds(0, 1)], out_hbm.at[0, pl.ds(0, 1)], sem_c).wait()


@jax.jit
def _rp2d(table_v, table_h):
    mesh = plsc.VectorSubcoreMesh(
        core_axis_name="c", subcore_axis_name="s",
        num_cores=_NC, num_subcores=_NS)
    return pl.kernel(
        _rp2d_body,
        out_type=jax.ShapeDtypeStruct((_N, _N, _D), jnp.float32),
        mesh=mesh,
        scratch_types=[
            pltpu.VMEM((2 * _S + 2, _D), jnp.float32),
            pltpu.VMEM((2 * _S + 2, _D), jnp.float32),
            pltpu.VMEM((_N, _D), jnp.float32),
            pltpu.VMEM((_S, _D), jnp.float32),
            pltpu.SemaphoreType.DMA,
            pltpu.SemaphoreType.DMA,
            pltpu.SemaphoreType.DMA,
        ],
    )(table_v, table_h)


def kernel(length_q, length_k, embeddings_table_v, embeddings_table_h):
    del length_q, length_k
    return _rp2d(embeddings_table_v, embeddings_table_h)


# speedup vs baseline: 1.5035x; 1.5035x over previous
import jax
import jax.numpy as jnp
from jax import lax
from jax.experimental import pallas as pl
from jax.experimental.pallas import tpu as pltpu
from jax.experimental.pallas import tpu_sc as plsc

_S = 24
_N = 577
_D = 64
_NV = _D // 16
_NC = 2
_NS = 16
_NW = _NC * _NS
_RPW = 18
_HA = 288


def _rp2d_body(v_hbm, h_hbm, out_hbm, v_vm, h_vm, row_vm, cls_vm,
               sem_a, sem_b, sem_c):
    w = lax.axis_index("s") * _NC + lax.axis_index("c")
    pltpu.sync_copy(v_hbm, v_vm)
    pltpu.sync_copy(h_hbm, h_vm)

    cls_vec = [v_vm[0, pl.ds(d * 16, 16)] + h_vm[0, pl.ds(d * 16, 16)]
               for d in range(_NV)]

    for r in range(_S):
        for d in range(_NV):
            cls_vm[r, pl.ds(d * 16, 16)] = cls_vec[d]

    @pl.when(w < _S)
    def _():
        pltpu.async_copy(cls_vm, out_hbm.at[0, pl.ds(w * _S, _S)], sem_c)

    @pl.when(w == _S)
    def _():
        pltpu.async_copy(cls_vm.at[pl.ds(0, 1)],
                         out_hbm.at[0, pl.ds(_N - 1, 1)], sem_c)

    def wait_half(sem, lo, n):
        pltpu.make_async_copy(
            row_vm.at[pl.ds(lo, n)], out_hbm.at[0, pl.ds(lo, n)], sem).wait()

    def _slice_starts(q):
        qb = (q - 1) // _S
        qr = (q - 1) % _S
        return (_S + 1) - qb, (_S + 1) - qr

    def _emit_group(vb, hb, kb0, nkb, kr_cut=None):
        vv = [[v_vm[vb + (kb0 + i), pl.ds(d * 16, 16)]
               for d in range(_NV)] for i in range(nkb)]
        for kr in range(_S):
            hrow = hb + kr
            for d in range(_NV):
                h = h_vm[hrow, pl.ds(d * 16, 16)]
                for i in range(nkb):
                    if kr_cut is not None and i == nkb - 1 and kr >= kr_cut:
                        continue
                    r = 1 + (kb0 + i) * _S + kr
                    row_vm[r, pl.ds(d * 16, 16)] = vv[i][d] + h

    def _emit_group_dyn(vb, hb, kb0):
        vv = [[v_vm[vb + kb0 + i, pl.ds(d * 16, 16)]
               for d in range(_NV)] for i in range(4)]
        rb = 1 + kb0 * _S
        for kr in range(_S):
            hrow = hb + kr
            for d in range(_NV):
                h = h_vm[hrow, pl.ds(d * 16, 16)]
                for i in range(4):
                    row_vm[rb + i * _S + kr, pl.ds(d * 16, 16)] = (
                        vv[i][d] + h)

    def do_row(j, carry):
        plsc.subcore_barrier()
        q = 1 + w + _NW * j
        vb, hb = _slice_starts(q)

        @pl.when(j >= 1)
        def _():
            wait_half(sem_a, 0, _HA)

        for d in range(_NV):
            row_vm[0, pl.ds(d * 16, 16)] = cls_vec[d]

        _emit_group(vb, hb, 0, 6)
        _emit_group(vb, hb, 6, 6, kr_cut=_S - 1)

        pltpu.async_copy(row_vm.at[pl.ds(0, _HA)],
                         out_hbm.at[q, pl.ds(0, _HA)], sem_a)

        @pl.when(j >= 1)
        def _():
            wait_half(sem_b, _HA, _N - _HA)

        for d in range(_NV):
            row_vm[_HA, pl.ds(d * 16, 16)] = (
                v_vm[vb + 11, pl.ds(d * 16, 16)]
                + h_vm[hb + _S - 1, pl.ds(d * 16, 16)])
        _emit_group(vb, hb, 12, 6)
        _emit_group(vb, hb, 18, 6)

        pltpu.async_copy(row_vm.at[pl.ds(_HA, _N - _HA)],
                         out_hbm.at[q, pl.ds(_HA, _N - _HA)], sem_b)

        return carry

    lax.fori_loop(0, _RPW, do_row, 0)
    wait_half(sem_a, 0, _HA)
    wait_half(sem_b, _HA, _N - _HA)

    @pl.when(w < _S)
    def _():
        pltpu.make_async_copy(
            cls_vm, out_hbm.at[0, pl.ds(0, _S)], sem_c).wait()

    @pl.when(w == _S)
    def _():
        pltpu.make_async_copy(
            cls_vm.at[pl.ds(0, 1)], out_hbm.at[0, pl.ds(0, 1)], sem_c).wait()


@jax.jit
def _rp2d(table_v, table_h):
    mesh = plsc.VectorSubcoreMesh(
        core_axis_name="c", subcore_axis_name="s",
        num_cores=_NC, num_subcores=_NS)
    return pl.kernel(
        _rp2d_body,
        out_type=jax.ShapeDtypeStruct((_N, _N, _D), jnp.float32),
        mesh=mesh,
        scratch_types=[
            pltpu.VMEM((2 * _S + 2, _D), jnp.float32),
            pltpu.VMEM((2 * _S + 2, _D), jnp.float32),
            pltpu.VMEM((_N, _D), jnp.float32),
            pltpu.VMEM((_S, _D), jnp.float32),
            pltpu.SemaphoreType.DMA,
            pltpu.SemaphoreType.DMA,
            pltpu.SemaphoreType.DMA,
        ],
    )(table_v, table_h)


def kernel(length_q, length_k, embeddings_table_v, embeddings_table_h):
    del length_q, length_k
    return _rp2d(embeddings_table_v, embeddings_table_h)


# speedup vs baseline: 1.5405x; 1.0246x over previous
import jax
import jax.numpy as jnp
from jax import lax
from jax.experimental import pallas as pl
from jax.experimental.pallas import tpu as pltpu
from jax.experimental.pallas import tpu_sc as plsc

_S = 24
_N = 577
_D = 64
_NV = _D // 16
_NC = 2
_NS = 16
_NW = _NC * _NS
_RPW = 18
_HA = 288


def _rp2d_body(v_hbm, h_hbm, out_hbm, v_vm, h_vm, row_vm, cls_vm,
               sem_a, sem_b, sem_c):
    w = lax.axis_index("s") * _NC + lax.axis_index("c")
    pltpu.sync_copy(v_hbm, v_vm)
    pltpu.sync_copy(h_hbm, h_vm)

    cls_vec = [v_vm[0, pl.ds(d * 16, 16)] + h_vm[0, pl.ds(d * 16, 16)]
               for d in range(_NV)]

    for r in range(_S):
        for d in range(_NV):
            cls_vm[r, pl.ds(d * 16, 16)] = cls_vec[d]

    @pl.when(w < _S)
    def _():
        pltpu.async_copy(cls_vm, out_hbm.at[0, pl.ds(w * _S, _S)], sem_c)

    @pl.when(w == _S)
    def _():
        pltpu.async_copy(cls_vm.at[pl.ds(0, 1)],
                         out_hbm.at[0, pl.ds(_N - 1, 1)], sem_c)

    def wait_half(sem, lo, n):
        pltpu.make_async_copy(
            row_vm.at[pl.ds(lo, n)], out_hbm.at[0, pl.ds(lo, n)], sem).wait()

    def _slice_starts(q):
        qb = (q - 1) // _S
        qr = (q - 1) % _S
        return (_S + 1) - qb, (_S + 1) - qr

    def _emit_group(vb, hb, kb0, nkb, kr_cut=None):
        vv = [[v_vm[vb + (kb0 + i), pl.ds(d * 16, 16)]
               for d in range(_NV)] for i in range(nkb)]
        for kr in range(_S):
            hrow = hb + kr
            for d in range(_NV):
                h = h_vm[hrow, pl.ds(d * 16, 16)]
                for i in range(nkb):
                    if kr_cut is not None and i == nkb - 1 and kr >= kr_cut:
                        continue
                    r = 1 + (kb0 + i) * _S + kr
                    row_vm[r, pl.ds(d * 16, 16)] = vv[i][d] + h

    def _emit_group_dyn(vb, hb, kb0):
        vv = [[v_vm[vb + kb0 + i, pl.ds(d * 16, 16)]
               for d in range(_NV)] for i in range(4)]
        rb = 1 + kb0 * _S
        for kr in range(_S):
            hrow = hb + kr
            for d in range(_NV):
                h = h_vm[hrow, pl.ds(d * 16, 16)]
                for i in range(4):
                    row_vm[rb + i * _S + kr, pl.ds(d * 16, 16)] = (
                        vv[i][d] + h)

    def do_row(j, carry):
        plsc.subcore_barrier()
        q = 1 + w + _NW * j
        vb, hb = _slice_starts(q)

        @pl.when(j >= 1)
        def _():
            wait_half(sem_a, 0, _HA)

        for d in range(_NV):
            row_vm[0, pl.ds(d * 16, 16)] = cls_vec[d]

        _emit_group(vb, hb, 0, 12, kr_cut=_S - 1)

        pltpu.async_copy(row_vm.at[pl.ds(0, _HA)],
                         out_hbm.at[q, pl.ds(0, _HA)], sem_a)

        @pl.when(j >= 1)
        def _():
            wait_half(sem_b, _HA, _N - _HA)

        for d in range(_NV):
            row_vm[_HA, pl.ds(d * 16, 16)] = (
                v_vm[vb + 11, pl.ds(d * 16, 16)]
                + h_vm[hb + _S - 1, pl.ds(d * 16, 16)])
        _emit_group(vb, hb, 12, 12)

        pltpu.async_copy(row_vm.at[pl.ds(_HA, _N - _HA)],
                         out_hbm.at[q, pl.ds(_HA, _N - _HA)], sem_b)

        return carry

    lax.fori_loop(0, _RPW, do_row, 0)
    wait_half(sem_a, 0, _HA)
    wait_half(sem_b, _HA, _N - _HA)

    @pl.when(w < _S)
    def _():
        pltpu.make_async_copy(
            cls_vm, out_hbm.at[0, pl.ds(0, _S)], sem_c).wait()

    @pl.when(w == _S)
    def _():
        pltpu.make_async_copy(
            cls_vm.at[pl.ds(0, 1)], out_hbm.at[0, pl.ds(0, 1)], sem_c).wait()


@jax.jit
def _rp2d(table_v, table_h):
    mesh = plsc.VectorSubcoreMesh(
        core_axis_name="c", subcore_axis_name="s",
        num_cores=_NC, num_subcores=_NS)
    return pl.kernel(
        _rp2d_body,
        out_type=jax.ShapeDtypeStruct((_N, _N, _D), jnp.float32),
        mesh=mesh,
        scratch_types=[
            pltpu.VMEM((2 * _S + 2, _D), jnp.float32),
            pltpu.VMEM((2 * _S + 2, _D), jnp.float32),
            pltpu.VMEM((_N, _D), jnp.float32),
            pltpu.VMEM((_S, _D), jnp.float32),
            pltpu.SemaphoreType.DMA,
            pltpu.SemaphoreType.DMA,
            pltpu.SemaphoreType.DMA,
        ],
    )(table_v, table_h)


def kernel(length_q, length_k, embeddings_table_v, embeddings_table_h):
    del length_q, length_k
    return _rp2d(embeddings_table_v, embeddings_table_h)


# speedup vs baseline: 1.5409x; 1.0003x over previous
import jax
import jax.numpy as jnp
from jax import lax
from jax.experimental import pallas as pl
from jax.experimental.pallas import tpu as pltpu
from jax.experimental.pallas import tpu_sc as plsc

_S = 24
_N = 577
_D = 64
_NV = _D // 16
_NC = 2
_NS = 16
_NW = _NC * _NS
_RPW = 18
_HA = 288


def _rp2d_body(v_hbm, h_hbm, out_hbm, v_vm, h_vm, row_vm, cls_vm,
               sem_a, sem_b, sem_c):
    w = lax.axis_index("s") * _NC + lax.axis_index("c")
    pltpu.sync_copy(v_hbm, v_vm)
    pltpu.sync_copy(h_hbm, h_vm)

    cls_vec = [v_vm[0, pl.ds(d * 16, 16)] + h_vm[0, pl.ds(d * 16, 16)]
               for d in range(_NV)]

    for r in range(_S):
        for d in range(_NV):
            cls_vm[r, pl.ds(d * 16, 16)] = cls_vec[d]

    @pl.when(w < _S)
    def _():
        pltpu.async_copy(cls_vm, out_hbm.at[0, pl.ds(w * _S, _S)], sem_c)

    @pl.when(w == _S)
    def _():
        pltpu.async_copy(cls_vm.at[pl.ds(0, 1)],
                         out_hbm.at[0, pl.ds(_N - 1, 1)], sem_c)

    def wait_half(sem, lo, n):
        pltpu.make_async_copy(
            row_vm.at[pl.ds(lo, n)], out_hbm.at[0, pl.ds(lo, n)], sem).wait()

    def _slice_starts(q):
        qb = (q - 1) // _S
        qr = (q - 1) % _S
        return (_S + 1) - qb, (_S + 1) - qr

    def _emit_group(vb, hb, kb0, nkb, kr_cut=None):
        vv = [[v_vm[vb + (kb0 + i), pl.ds(d * 16, 16)]
               for d in range(_NV)] for i in range(nkb)]
        for kr in range(_S):
            hrow = hb + kr
            for d in range(_NV):
                h = h_vm[hrow, pl.ds(d * 16, 16)]
                for i in range(nkb):
                    if kr_cut is not None and i == nkb - 1 and kr >= kr_cut:
                        continue
                    r = 1 + (kb0 + i) * _S + kr
                    row_vm[r, pl.ds(d * 16, 16)] = vv[i][d] + h

    def do_row(j, carry):
        plsc.subcore_barrier()
        q = 1 + w + _NW * j
        vb, hb = _slice_starts(q)

        @pl.when(j >= 1)
        def _():
            wait_half(sem_a, 0, _HA)

        for d in range(_NV):
            row_vm[0, pl.ds(d * 16, 16)] = cls_vec[d]

        _emit_group(vb, hb, 0, 12, kr_cut=_S - 1)

        pltpu.async_copy(row_vm.at[pl.ds(0, _HA)],
                         out_hbm.at[q, pl.ds(0, _HA)], sem_a)

        @pl.when(j >= 1)
        def _():
            wait_half(sem_b, _HA, _N - _HA)

        for d in range(_NV):
            row_vm[_HA, pl.ds(d * 16, 16)] = (
                v_vm[vb + 11, pl.ds(d * 16, 16)]
                + h_vm[hb + _S - 1, pl.ds(d * 16, 16)])
        _emit_group(vb, hb, 12, 12)

        pltpu.async_copy(row_vm.at[pl.ds(_HA, _N - _HA)],
                         out_hbm.at[q, pl.ds(_HA, _N - _HA)], sem_b)

        return carry

    lax.fori_loop(0, _RPW, do_row, 0)
    wait_half(sem_a, 0, _HA)
    wait_half(sem_b, _HA, _N - _HA)

    @pl.when(w < _S)
    def _():
        pltpu.make_async_copy(
            cls_vm, out_hbm.at[0, pl.ds(0, _S)], sem_c).wait()

    @pl.when(w == _S)
    def _():
        pltpu.make_async_copy(
            cls_vm.at[pl.ds(0, 1)], out_hbm.at[0, pl.ds(0, 1)], sem_c).wait()


@jax.jit
def _rp2d(table_v, table_h):
    mesh = plsc.VectorSubcoreMesh(
        core_axis_name="c", subcore_axis_name="s",
        num_cores=_NC, num_subcores=_NS)
    return pl.kernel(
        _rp2d_body,
        out_type=jax.ShapeDtypeStruct((_N, _N, _D), jnp.float32),
        mesh=mesh,
        scratch_types=[
            pltpu.VMEM((2 * _S + 2, _D), jnp.float32),
            pltpu.VMEM((2 * _S + 2, _D), jnp.float32),
            pltpu.VMEM((_N, _D), jnp.float32),
            pltpu.VMEM((_S, _D), jnp.float32),
            pltpu.SemaphoreType.DMA,
            pltpu.SemaphoreType.DMA,
            pltpu.SemaphoreType.DMA,
        ],
    )(table_v, table_h)


def kernel(length_q, length_k, embeddings_table_v, embeddings_table_h):
    del length_q, length_k
    return _rp2d(embeddings_table_v, embeddings_table_h)


# speedup vs baseline: 1.5673x; 1.0172x over previous
import jax
import jax.numpy as jnp
from jax import lax
from jax.experimental import pallas as pl
from jax.experimental.pallas import tpu as pltpu
from jax.experimental.pallas import tpu_sc as plsc

_S = 24
_N = 577
_D = 64
_NV = _D // 16
_NC = 2
_NS = 16
_NW = _NC * _NS
_RPW = 18
_HA = 288


def _rp2d_body(v_hbm, h_hbm, out_hbm, v_vm, h_vm, row_vm, cls_vm,
               sem_a, sem_b, sem_c):
    w = lax.axis_index("s") * _NC + lax.axis_index("c")
    pltpu.sync_copy(v_hbm, v_vm)
    pltpu.sync_copy(h_hbm, h_vm)

    cls_vec = [v_vm[0, pl.ds(d * 16, 16)] + h_vm[0, pl.ds(d * 16, 16)]
               for d in range(_NV)]

    for r in range(_S):
        for d in range(_NV):
            cls_vm[r, pl.ds(d * 16, 16)] = cls_vec[d]

    @pl.when(w < _S)
    def _():
        pltpu.async_copy(cls_vm, out_hbm.at[0, pl.ds(w * _S, _S)], sem_c)

    @pl.when(w == _S)
    def _():
        pltpu.async_copy(cls_vm.at[pl.ds(0, 1)],
                         out_hbm.at[0, pl.ds(_N - 1, 1)], sem_c)

    def wait_half(sem, lo, n):
        pltpu.make_async_copy(
            row_vm.at[pl.ds(lo, n)], out_hbm.at[0, pl.ds(lo, n)], sem).wait()

    def _slice_starts(q):
        qb = (q - 1) // _S
        qr = (q - 1) % _S
        return (_S + 1) - qb, (_S + 1) - qr

    def _emit_group(vb, hb, kb0, nkb, kr_cut=None):
        vv = [[v_vm[vb + (kb0 + i), pl.ds(d * 16, 16)]
               for d in range(_NV)] for i in range(nkb)]
        for kr in range(_S):
            hrow = hb + kr
            for d in range(_NV):
                h = h_vm[hrow, pl.ds(d * 16, 16)]
                for i in range(nkb):
                    if kr_cut is not None and i == nkb - 1 and kr >= kr_cut:
                        continue
                    r = 1 + (kb0 + i) * _S + kr
                    row_vm[r, pl.ds(d * 16, 16)] = vv[i][d] + h

    def do_row(j, carry):
        plsc.subcore_barrier()
        q = 1 + w + _NW * j
        vb, hb = _slice_starts(q)

        @pl.when(j >= 1)
        def _():
            wait_half(sem_a, 0, _HA)

        for d in range(_NV):
            row_vm[0, pl.ds(d * 16, 16)] = cls_vec[d]

        _emit_group(vb, hb, 0, 12, kr_cut=_S - 1)

        pltpu.async_copy(row_vm.at[pl.ds(0, _HA)],
                         out_hbm.at[q, pl.ds(0, _HA)], sem_a)

        plsc.subcore_barrier()
        @pl.when(j >= 1)
        def _():
            wait_half(sem_b, _HA, _N - _HA)

        for d in range(_NV):
            row_vm[_HA, pl.ds(d * 16, 16)] = (
                v_vm[vb + 11, pl.ds(d * 16, 16)]
                + h_vm[hb + _S - 1, pl.ds(d * 16, 16)])
        _emit_group(vb, hb, 12, 12)

        pltpu.async_copy(row_vm.at[pl.ds(_HA, _N - _HA)],
                         out_hbm.at[q, pl.ds(_HA, _N - _HA)], sem_b)

        return carry

    lax.fori_loop(0, _RPW, do_row, 0)
    wait_half(sem_a, 0, _HA)
    wait_half(sem_b, _HA, _N - _HA)

    @pl.when(w < _S)
    def _():
        pltpu.make_async_copy(
            cls_vm, out_hbm.at[0, pl.ds(0, _S)], sem_c).wait()

    @pl.when(w == _S)
    def _():
        pltpu.make_async_copy(
            cls_vm.at[pl.ds(0, 1)], out_hbm.at[0, pl.ds(0, 1)], sem_c).wait()


@jax.jit
def _rp2d(table_v, table_h):
    mesh = plsc.VectorSubcoreMesh(
        core_axis_name="c", subcore_axis_name="s",
        num_cores=_NC, num_subcores=_NS)
    return pl.kernel(
        _rp2d_body,
        out_type=jax.ShapeDtypeStruct((_N, _N, _D), jnp.float32),
        mesh=mesh,
        scratch_types=[
            pltpu.VMEM((2 * _S + 2, _D), jnp.float32),
            pltpu.VMEM((2 * _S + 2, _D), jnp.float32),
            pltpu.VMEM((_N, _D), jnp.float32),
            pltpu.VMEM((_S, _D), jnp.float32),
            pltpu.SemaphoreType.DMA,
            pltpu.SemaphoreType.DMA,
            pltpu.SemaphoreType.DMA,
        ],
    )(table_v, table_h)


def kernel(length_q, length_k, embeddings_table_v, embeddings_table_h):
    del length_q, length_k
    return _rp2d(embeddings_table_v, embeddings_table_h)


# speedup vs baseline: 1.6180x; 1.0324x over previous
import jax
import jax.numpy as jnp
from jax import lax
from jax.experimental import pallas as pl
from jax.experimental.pallas import tpu as pltpu
from jax.experimental.pallas import tpu_sc as plsc

_S = 24
_N = 577
_D = 64
_NV = _D // 16
_NC = 2
_NS = 16
_NW = _NC * _NS
_RPW = 18
_HA = 288


def _rp2d_body(v_hbm, h_hbm, out_hbm, v_vm, h_vm, row_vm, cls_vm,
               sem_a, sem_b, sem_c):
    w = lax.axis_index("s") * _NC + lax.axis_index("c")
    pltpu.sync_copy(v_hbm, v_vm)
    pltpu.sync_copy(h_hbm, h_vm)

    cls_vec = [v_vm[0, pl.ds(d * 16, 16)] + h_vm[0, pl.ds(d * 16, 16)]
               for d in range(_NV)]

    for r in range(_S):
        for d in range(_NV):
            cls_vm[r, pl.ds(d * 16, 16)] = cls_vec[d]

    @pl.when(w < _S)
    def _():
        pltpu.async_copy(cls_vm, out_hbm.at[0, pl.ds(w * _S, _S)], sem_c)

    @pl.when(w == _S)
    def _():
        pltpu.async_copy(cls_vm.at[pl.ds(0, 1)],
                         out_hbm.at[0, pl.ds(_N - 1, 1)], sem_c)

    def wait_half(sem, lo, n):
        pltpu.make_async_copy(
            row_vm.at[pl.ds(lo, n)], out_hbm.at[0, pl.ds(lo, n)], sem).wait()

    def _slice_starts(q):
        qb = (q - 1) // _S
        qr = (q - 1) % _S
        return (_S + 1) - qb, (_S + 1) - qr

    def _emit_group(vb, hb, kb0, nkb, kr_cut=None):
        vv = [[v_vm[vb + (kb0 + i), pl.ds(d * 16, 16)]
               for d in range(_NV)] for i in range(nkb)]
        for kr in range(_S):
            if kr in (8, 16):
                plsc.subcore_barrier()
            hrow = hb + kr
            for d in range(_NV):
                h = h_vm[hrow, pl.ds(d * 16, 16)]
                for i in range(nkb):
                    if kr_cut is not None and i == nkb - 1 and kr >= kr_cut:
                        continue
                    r = 1 + (kb0 + i) * _S + kr
                    row_vm[r, pl.ds(d * 16, 16)] = vv[i][d] + h

    def do_row(j, carry):
        plsc.subcore_barrier()
        q = 1 + w + _NW * j
        vb, hb = _slice_starts(q)

        @pl.when(j >= 1)
        def _():
            wait_half(sem_a, 0, _HA)

        for d in range(_NV):
            row_vm[0, pl.ds(d * 16, 16)] = cls_vec[d]

        _emit_group(vb, hb, 0, 12, kr_cut=_S - 1)

        pltpu.async_copy(row_vm.at[pl.ds(0, _HA)],
                         out_hbm.at[q, pl.ds(0, _HA)], sem_a)

        plsc.subcore_barrier()
        @pl.when(j >= 1)
        def _():
            wait_half(sem_b, _HA, _N - _HA)

        for d in range(_NV):
            row_vm[_HA, pl.ds(d * 16, 16)] = (
                v_vm[vb + 11, pl.ds(d * 16, 16)]
                + h_vm[hb + _S - 1, pl.ds(d * 16, 16)])
        _emit_group(vb, hb, 12, 12)

        pltpu.async_copy(row_vm.at[pl.ds(_HA, _N - _HA)],
                         out_hbm.at[q, pl.ds(_HA, _N - _HA)], sem_b)

        return carry

    lax.fori_loop(0, _RPW, do_row, 0)
    wait_half(sem_a, 0, _HA)
    wait_half(sem_b, _HA, _N - _HA)

    @pl.when(w < _S)
    def _():
        pltpu.make_async_copy(
            cls_vm, out_hbm.at[0, pl.ds(0, _S)], sem_c).wait()

    @pl.when(w == _S)
    def _():
        pltpu.make_async_copy(
            cls_vm.at[pl.ds(0, 1)], out_hbm.at[0, pl.ds(0, 1)], sem_c).wait()


@jax.jit
def _rp2d(table_v, table_h):
    mesh = plsc.VectorSubcoreMesh(
        core_axis_name="c", subcore_axis_name="s",
        num_cores=_NC, num_subcores=_NS)
    return pl.kernel(
        _rp2d_body,
        out_type=jax.ShapeDtypeStruct((_N, _N, _D), jnp.float32),
        mesh=mesh,
        scratch_types=[
            pltpu.VMEM((2 * _S + 2, _D), jnp.float32),
            pltpu.VMEM((2 * _S + 2, _D), jnp.float32),
            pltpu.VMEM((_N, _D), jnp.float32),
            pltpu.VMEM((_S, _D), jnp.float32),
            pltpu.SemaphoreType.DMA,
            pltpu.SemaphoreType.DMA,
            pltpu.SemaphoreType.DMA,
        ],
    )(table_v, table_h)


def kernel(length_q, length_k, embeddings_table_v, embeddings_table_h):
    del length_q, length_k
    return _rp2d(embeddings_table_v, embeddings_table_h)
